# baseline (speedup 1.0000x reference)
import jax
import jax.numpy as jnp
from jax.experimental import pallas as pl
from jax.experimental.pallas import tpu as pltpu

_E, _CAP, _D = 64, 1024, 512
_H = 4 * _D
_RC = 256
_NC = _CAP // _RC


def _gelu(v):
    return 0.5 * v * (1.0 + jax.lax.erf(v * 0.7071067811865476))


def _ffn_body(sp_ref, x0, x1, x2, x3, w1a, w1b, b1a, b1b, w2a, w2b, b2_ref, o_ref):
    e = pl.program_id(0)
    valid = sp_ref[0, e]
    xs = (x0, x1, x2, x3)

    for k in range(_NC):
        base = k * _RC
        rows = slice(base, base + _RC)

        @pl.when(base < valid)
        def _compute(rows=rows, base=base, x_ref=xs[k]):
            x = x_ref[0, 0]
            ha = _gelu(jnp.dot(x, w1a[0], preferred_element_type=jnp.float32) + b1a[0])
            hb = _gelu(jnp.dot(x, w1b[0], preferred_element_type=jnp.float32) + b1b[0])
            y = (
                jnp.dot(ha, w2a[0], preferred_element_type=jnp.float32)
                + jnp.dot(hb, w2b[0], preferred_element_type=jnp.float32)
                + b2_ref[0]
            )
            ridx = base + jax.lax.broadcasted_iota(jnp.int32, (_RC, 1), 0)
            o_ref[0, rows, :] = jnp.where(ridx < valid, y, 0.0)

        @pl.when(base >= valid)
        def _zero(rows=rows):
            o_ref[0, rows, :] = jnp.zeros((_RC, _D), jnp.float32)


def kernel(packed_inputs, valid_load, w1, b1, w2, b2):
    vl = valid_load.astype(jnp.int32)

    eids = jnp.arange(_E, dtype=jnp.int32)
    thresh = jnp.array([k * _RC for k in range(_NC)], jnp.int32)
    chunk_valid = vl[None, :] > thresh[:, None]
    any_valid = (vl > 0)[None, :]
    live = jnp.concatenate([chunk_valid, any_valid], axis=0)
    maps = jax.lax.cummax(jnp.where(live, eids[None, :], 0), axis=1)
    sp = jnp.concatenate([vl[None, :], maps], axis=0)

    xr = packed_inputs.reshape(_E, _NC, _RC, _D)
    b1r = b1.reshape(_E, 1, _H)
    b2r = b2.reshape(_E, 1, _D)

    def _xmap(k):
        return lambda e, sp_ref: (sp_ref[1 + k, e], k, 0, 0)

    def _wmap(e, sp_ref):
        return (sp_ref[1 + _NC, e], 0, 0)

    out = pl.pallas_call(
        _ffn_body,
        out_shape=jax.ShapeDtypeStruct((_E, _CAP, _D), jnp.float32),
        grid_spec=pltpu.PrefetchScalarGridSpec(
            num_scalar_prefetch=1,
            grid=(_E,),
            in_specs=[
                pl.BlockSpec((1, 1, _RC, _D), _xmap(0)),
                pl.BlockSpec((1, 1, _RC, _D), _xmap(1)),
                pl.BlockSpec((1, 1, _RC, _D), _xmap(2)),
                pl.BlockSpec((1, 1, _RC, _D), _xmap(3)),
                pl.BlockSpec((1, _D, _H // 2), lambda e, sp: (sp[1 + _NC, e], 0, 0)),
                pl.BlockSpec((1, _D, _H // 2), lambda e, sp: (sp[1 + _NC, e], 0, 1)),
                pl.BlockSpec((1, 1, _H // 2), lambda e, sp: (sp[1 + _NC, e], 0, 0)),
                pl.BlockSpec((1, 1, _H // 2), lambda e, sp: (sp[1 + _NC, e], 0, 1)),
                pl.BlockSpec((1, _H // 2, _D), lambda e, sp: (sp[1 + _NC, e], 0, 0)),
                pl.BlockSpec((1, _H // 2, _D), lambda e, sp: (sp[1 + _NC, e], 1, 0)),
                pl.BlockSpec((1, 1, _D), _wmap),
            ],
            out_specs=pl.BlockSpec((1, _CAP, _D), lambda e, sp_ref: (e, 0, 0)),
        ),
        compiler_params=pltpu.CompilerParams(
            dimension_semantics=("parallel",),
            vmem_limit_bytes=56 * 1024 * 1024,
        ),
        name="fused_group_ffn",
    )(sp, xr, xr, xr, xr, w1, w1, b1r, b1r, w2, w2, b2r)
    return out
